# baseline (speedup 1.0000x reference)
import functools

import jax
import jax.numpy as jnp
from jax import lax
from jax.experimental import pallas as pl
from jax.experimental.pallas import tpu as pltpu
from jax.experimental.pallas import tpu_sc as plsc

N = 10000
M = 5000
NPAD = 10240
K = 32
DIN = 128
D1 = 128
D2 = 64
E = 320000

NC = 2
NS = 16
NW = NC * NS
NPW = NPAD // NW
HALF = NPAD // NC
CHN = 4
CHK = CHN * K
NCHH = NPW // CHN
EPW = E // NW
ECH = 40
NECH = EPW // ECH

_mesh = plsc.VectorSubcoreMesh(core_axis_name="c", subcore_axis_name="s")


def _zero_rows(zbuf, rows, width):
    for r in range(rows):
        for v in range(width // 16):
            zbuf[r, pl.ds(v * 16, 16)] = jnp.zeros((16,), jnp.float32)


@functools.partial(
    pl.kernel,
    mesh=_mesh,
    out_type=jax.ShapeDtypeStruct((NPAD, DIN), jnp.float32),
    scratch_types=[
        pltpu.VMEM((NPW * K,), jnp.int32),
        pltpu.VMEM((2, CHK, DIN), jnp.float32),
        pltpu.VMEM((NPW, DIN), jnp.float32),
        pltpu.SemaphoreType.DMA,
    ],
)
def _hop_gather(table_hbm, idx_hbm, out_hbm, idx_v, gbuf, obuf, gsem):
    c = lax.axis_index("c")
    s = lax.axis_index("s")
    wid = c * NS + s
    pltpu.sync_copy(idx_hbm.at[pl.ds(wid * NPW * K, NPW * K)], idx_v)
    pltpu.async_copy(table_hbm.at[idx_v.at[pl.ds(0, CHK)]], gbuf.at[0], gsem)

    def body(gp, carry):
        for b in range(2):
            g = 2 * gp + b

            @pl.when(g + 1 < NCHH)
            def _():
                pltpu.async_copy(
                    table_hbm.at[idx_v.at[pl.ds((g + 1) * CHK, CHK)]],
                    gbuf.at[1 - b], gsem)

            pltpu.make_async_copy(
                table_hbm.at[pl.ds(0, CHK)], gbuf.at[b], gsem).wait()
            for j in range(CHN):
                for v in range(DIN // 16):
                    col = pl.ds(v * 16, 16)
                    a0 = gbuf[b, j * K + 0, col]
                    a1 = gbuf[b, j * K + 1, col]
                    a2 = gbuf[b, j * K + 2, col]
                    a3 = gbuf[b, j * K + 3, col]
                    for k in range(4, K, 4):
                        a0 = a0 + gbuf[b, j * K + k + 0, col]
                        a1 = a1 + gbuf[b, j * K + k + 1, col]
                        a2 = a2 + gbuf[b, j * K + k + 2, col]
                        a3 = a3 + gbuf[b, j * K + k + 3, col]
                    obuf[g * CHN + j, col] = (a0 + a1) + (a2 + a3)
        return carry

    lax.fori_loop(0, NCHH // 2, body, 0)
    pltpu.sync_copy(obuf, out_hbm.at[pl.ds(wid * NPW, NPW)])


@functools.partial(
    pl.kernel,
    mesh=_mesh,
    out_type=jax.ShapeDtypeStruct((NC, NPAD), jnp.float32),
    scratch_types=[
        pltpu.VMEM((NECH, ECH), jnp.int32),
        pltpu.VMEM((ECH,), jnp.float32),
        pltpu.VMEM((NPAD // NS,), jnp.float32),
        pltpu.VMEM_SHARED((NPAD,), jnp.float32),
    ],
)
def _deg_count(dst_hbm, out_hbm, dstv, ones_v, zb, acc):
    c = lax.axis_index("c")
    s = lax.axis_index("s")
    wid = c * NS + s
    zslice = NPAD // NS
    pltpu.sync_copy(dst_hbm.at[wid], dstv)
    for v in range(ECH // 16):
        ones_v[pl.ds(v * 16, 16)] = jnp.ones((16,), jnp.float32)
    for v in range(zslice // 16):
        zb[pl.ds(v * 16, 16)] = jnp.zeros((16,), jnp.float32)
    pltpu.sync_copy(zb, acc.at[pl.ds(s * zslice, zslice)])
    plsc.subcore_barrier()

    def body(g, carry):
        pltpu.sync_copy(ones_v, acc.at[dstv.at[g]], add=True)
        return carry

    lax.fori_loop(0, NECH, body, 0)
    plsc.subcore_barrier()
    pltpu.sync_copy(acc.at[pl.ds(s * zslice, zslice)],
                    out_hbm.at[c, pl.ds(s * zslice, zslice)])


SEG = 2000
NSEG = EPW // SEG
NCHS = SEG // ECH


@functools.partial(
    pl.kernel,
    mesh=_mesh,
    out_type=jax.ShapeDtypeStruct((NC, NPAD, DIN), jnp.float32),
    scratch_types=[
        pltpu.VMEM((SEG,), jnp.int32),
        pltpu.VMEM((NECH, ECH), jnp.int32),
        pltpu.VMEM((2, ECH, DIN), jnp.float32),
        pltpu.VMEM((16, DIN), jnp.float32),
        pltpu.VMEM_SHARED((NPAD, DIN), jnp.float32),
        pltpu.SemaphoreType.DMA,
    ],
)
def _gcn_msg(y_hbm, src_hbm, dst_hbm, out_hbm, srcv, dstv, gbuf, zbuf, acc,
             gsem):
    c = lax.axis_index("c")
    s = lax.axis_index("s")
    wid = c * NS + s
    zslice = NPAD // NS
    _zero_rows(zbuf, 16, DIN)
    for t in range(zslice // 16):
        pltpu.sync_copy(zbuf, acc.at[pl.ds(s * zslice + t * 16, 16)])
    pltpu.sync_copy(dst_hbm.at[wid], dstv)
    plsc.subcore_barrier()

    def seg_body(q, carry0):
        pltpu.sync_copy(src_hbm.at[pl.ds(wid * EPW + q * SEG, SEG)], srcv)
        pltpu.async_copy(y_hbm.at[srcv.at[pl.ds(0, ECH)]], gbuf.at[0], gsem)

        def body(gp, carry):
            for b in range(2):
                g = 2 * gp + b

                @pl.when(g + 1 < NCHS)
                def _():
                    pltpu.async_copy(
                        y_hbm.at[srcv.at[pl.ds((g + 1) * ECH, ECH)]],
                        gbuf.at[1 - b], gsem)

                pltpu.make_async_copy(
                    y_hbm.at[pl.ds(0, ECH)], gbuf.at[b], gsem).wait()
                pltpu.sync_copy(gbuf.at[b],
                                acc.at[dstv.at[q * NCHS + g]], add=True)
            return carry

        lax.fori_loop(0, NCHS // 2, body, 0)
        return carry0

    lax.fori_loop(0, NSEG, seg_body, 0)
    plsc.subcore_barrier()
    pltpu.sync_copy(acc.at[pl.ds(s * zslice, zslice)],
                    out_hbm.at[c, pl.ds(s * zslice, zslice)])


def _dense_hop(x, w):
    def body(x_ref, w_ref, o_ref):
        h = jnp.dot(x_ref[...] * (1.0 / K), w_ref[...],
                    preferred_element_type=jnp.float32)
        o_ref[...] = jnp.where(h > 0, h, 0.01 * h)

    return pl.pallas_call(
        body,
        grid=(NPAD // 256,),
        in_specs=[pl.BlockSpec((256, DIN), lambda i: (i, 0)),
                  pl.BlockSpec((DIN, D1), lambda i: (0, 0))],
        out_specs=pl.BlockSpec((256, D1), lambda i: (i, 0)),
        out_shape=jax.ShapeDtypeStruct((NPAD, D1), jnp.float32),
    )(x, w)


def _gcn_pre(deg_t, x):
    def body(d_ref, x_ref, y_ref, dv_ref):
        deg = jnp.maximum(d_ref[:, 0:1] + d_ref[:, 1:2], 1.0)
        dinv = 1.0 / jnp.sqrt(deg)
        dv_ref[...] = dinv
        y_ref[...] = x_ref[...] * dinv

    return pl.pallas_call(
        body,
        grid=(NPAD // 256,),
        in_specs=[pl.BlockSpec((256, NC), lambda i: (i, 0)),
                  pl.BlockSpec((256, DIN), lambda i: (i, 0))],
        out_specs=[pl.BlockSpec((256, DIN), lambda i: (i, 0)),
                   pl.BlockSpec((256, 1), lambda i: (i, 0))],
        out_shape=[jax.ShapeDtypeStruct((NPAD, DIN), jnp.float32),
                   jax.ShapeDtypeStruct((NPAD, 1), jnp.float32)],
    )(deg_t, x)


def _att_pair(h1, h2, a_ref):
    s1 = jnp.dot(h1, a_ref[...], preferred_element_type=jnp.float32)
    s2 = jnp.dot(h2, a_ref[...], preferred_element_type=jnp.float32)
    m = jnp.maximum(s1, s2)
    e1 = jnp.exp(s1 - m)
    e2 = jnp.exp(s2 - m)
    inv = 1.0 / (e1 + e2)
    return (e1 * inv) * h1 + (e2 * inv) * h2


def _post(parts, dinv, c1, c2, gcn_w, gcn_b, at2_w, at2_a, atb_w, atb_a):
    def body(gp_ref, dv_ref, c1_ref, c2_ref, gw_ref, gb_ref, aw_ref,
             aa_ref, bw_ref, ba_ref, o_ref):
        agg = (gp_ref[0] + gp_ref[1]) * dv_ref[...]
        g = jnp.maximum(
            jnp.dot(agg, gw_ref[...], preferred_element_type=jnp.float32)
            + gb_ref[...], 0.0)
        h1 = jnp.tanh(jnp.dot(c1_ref[...], aw_ref[...],
                              preferred_element_type=jnp.float32))
        h2 = jnp.tanh(jnp.dot(c2_ref[...], aw_ref[...],
                              preferred_element_type=jnp.float32))
        a1 = _att_pair(h1, h2, aa_ref)
        hb1 = jnp.tanh(jnp.dot(a1, bw_ref[...],
                               preferred_element_type=jnp.float32))
        hb2 = jnp.tanh(jnp.dot(g, bw_ref[...],
                               preferred_element_type=jnp.float32))
        o_ref[...] = _att_pair(hb1, hb2, ba_ref)

    return pl.pallas_call(
        body,
        grid=(NPAD // 256,),
        in_specs=[pl.BlockSpec((NC, 256, DIN), lambda i: (0, i, 0)),
                  pl.BlockSpec((256, 1), lambda i: (i, 0)),
                  pl.BlockSpec((256, D1), lambda i: (i, 0)),
                  pl.BlockSpec((256, D1), lambda i: (i, 0)),
                  pl.BlockSpec((DIN, D2), lambda i: (0, 0)),
                  pl.BlockSpec((1, D2), lambda i: (0, 0)),
                  pl.BlockSpec((D1, D2), lambda i: (0, 0)),
                  pl.BlockSpec((D2, 1), lambda i: (0, 0)),
                  pl.BlockSpec((D2, D2), lambda i: (0, 0)),
                  pl.BlockSpec((D2, 1), lambda i: (0, 0))],
        out_specs=pl.BlockSpec((256, D2), lambda i: (i, 0)),
        out_shape=jax.ShapeDtypeStruct((NPAD, D2), jnp.float32),
    )(parts, dinv, c1, c2, gcn_w, gcn_b, at2_w, at2_a, atb_w, atb_a)


def _score(mi, di):
    def body(a_ref, b_ref, o_ref):
        sc = lax.dot_general(a_ref[...], b_ref[...],
                             (((1,), (1,)), ((), ())),
                             preferred_element_type=jnp.float32)
        o_ref[...] = 1.0 / (1.0 + jnp.exp(-sc))

    return pl.pallas_call(
        body,
        grid=(5,),
        in_specs=[pl.BlockSpec((1000, D2), lambda i: (i, 0)),
                  pl.BlockSpec((M, D2), lambda i: (0, 0))],
        out_specs=pl.BlockSpec((1000, M), lambda i: (i, 0)),
        out_shape=jax.ShapeDtypeStruct((M, M), jnp.float32),
    )(mi, di)


def kernel(channel_1_k_neighbors, channel_2_k_neighbors,
           md_adj_withsl_edge_index, node_table, ch1_W, ch2_W, gcn_W, gcn_b,
           at2_W, at2_a, atb_W, atb_a):
    table = jnp.pad(node_table, ((0, NPAD - N), (0, 0)))
    idx1 = jnp.pad(channel_1_k_neighbors.astype(jnp.int32),
                   ((0, NPAD - N), (0, 0))).reshape(-1)
    idx2 = jnp.pad(channel_2_k_neighbors.astype(jnp.int32),
                   ((0, NPAD - N), (0, 0))).reshape(-1)
    src = md_adj_withsl_edge_index[0].astype(jnp.int32)
    dst3 = md_adj_withsl_edge_index[1].astype(jnp.int32).reshape(
        NW, NECH, ECH)

    h1 = table
    for i in range(3):
        h1 = _dense_hop(_hop_gather(h1, idx1), ch1_W[i])
    h2 = table
    for i in range(4):
        h2 = _dense_hop(_hop_gather(h2, idx2), ch2_W[i])

    degp = _deg_count(dst3)
    y, dinv = _gcn_pre(degp.T, table)
    parts = _gcn_msg(y, src, dst3)

    embed = _post(parts, dinv, h1, h2, gcn_W,
                  jnp.reshape(gcn_b, (1, D2)), at2_W,
                  jnp.reshape(at2_a, (D2, 1)), atb_W,
                  jnp.reshape(atb_a, (D2, 1)))
    return _score(embed[:M], embed[M:N])


# speedup vs baseline: 1.0131x; 1.0131x over previous
import functools

import jax
import jax.numpy as jnp
from jax import lax
from jax.experimental import pallas as pl
from jax.experimental.pallas import tpu as pltpu
from jax.experimental.pallas import tpu_sc as plsc

N = 10000
M = 5000
NPAD = 10240
K = 32
DIN = 128
D1 = 128
D2 = 64
E = 320000

NC = 2
NS = 16
NW = NC * NS
NPW = NPAD // NW
HALF = NPAD // NC
CHN = 2
CHK = CHN * K
NCHH = NPW // CHN
NBUF = 4
EPW = E // NW
ECH = 40
NECH = EPW // ECH

_mesh = plsc.VectorSubcoreMesh(core_axis_name="c", subcore_axis_name="s")


def _zero_rows(zbuf, rows, width):
    for r in range(rows):
        for v in range(width // 16):
            zbuf[r, pl.ds(v * 16, 16)] = jnp.zeros((16,), jnp.float32)


@functools.partial(
    pl.kernel,
    mesh=_mesh,
    out_type=jax.ShapeDtypeStruct((NPAD, DIN), jnp.float32),
    scratch_types=[
        pltpu.VMEM((NPW * K,), jnp.int32),
        pltpu.VMEM((NBUF, CHK, DIN), jnp.float32),
        pltpu.VMEM((NPW, DIN), jnp.float32),
        pltpu.SemaphoreType.DMA,
    ],
)
def _hop_gather(table_hbm, idx_hbm, out_hbm, idx_v, gbuf, obuf, gsem):
    c = lax.axis_index("c")
    s = lax.axis_index("s")
    wid = c * NS + s
    pltpu.sync_copy(idx_hbm.at[pl.ds(wid * NPW * K, NPW * K)], idx_v)
    for b in range(NBUF):
        pltpu.async_copy(table_hbm.at[idx_v.at[pl.ds(b * CHK, CHK)]],
                         gbuf.at[b], gsem)

    def body(gp, carry):
        for b in range(NBUF):
            g = NBUF * gp + b

            pltpu.make_async_copy(
                table_hbm.at[pl.ds(0, CHK)], gbuf.at[b], gsem).wait()
            for j in range(CHN):
                for v in range(DIN // 16):
                    col = pl.ds(v * 16, 16)
                    a0 = gbuf[b, j * K + 0, col]
                    a1 = gbuf[b, j * K + 1, col]
                    a2 = gbuf[b, j * K + 2, col]
                    a3 = gbuf[b, j * K + 3, col]
                    for k in range(4, K, 4):
                        a0 = a0 + gbuf[b, j * K + k + 0, col]
                        a1 = a1 + gbuf[b, j * K + k + 1, col]
                        a2 = a2 + gbuf[b, j * K + k + 2, col]
                        a3 = a3 + gbuf[b, j * K + k + 3, col]
                    obuf[g * CHN + j, col] = (a0 + a1) + (a2 + a3)

            @pl.when(g + NBUF < NCHH)
            def _():
                pltpu.async_copy(
                    table_hbm.at[idx_v.at[pl.ds((g + NBUF) * CHK, CHK)]],
                    gbuf.at[b], gsem)
        return carry

    lax.fori_loop(0, NCHH // NBUF, body, 0)
    pltpu.sync_copy(obuf, out_hbm.at[pl.ds(wid * NPW, NPW)])


@functools.partial(
    pl.kernel,
    mesh=_mesh,
    out_type=jax.ShapeDtypeStruct((NC, NPAD), jnp.float32),
    scratch_types=[
        pltpu.VMEM((NECH, ECH), jnp.int32),
        pltpu.VMEM((ECH,), jnp.float32),
        pltpu.VMEM((NPAD // NS,), jnp.float32),
        pltpu.VMEM_SHARED((NPAD,), jnp.float32),
    ],
)
def _deg_count(dst_hbm, out_hbm, dstv, ones_v, zb, acc):
    c = lax.axis_index("c")
    s = lax.axis_index("s")
    wid = c * NS + s
    zslice = NPAD // NS
    pltpu.sync_copy(dst_hbm.at[wid], dstv)
    for v in range(ECH // 16):
        ones_v[pl.ds(v * 16, 16)] = jnp.ones((16,), jnp.float32)
    for v in range(zslice // 16):
        zb[pl.ds(v * 16, 16)] = jnp.zeros((16,), jnp.float32)
    pltpu.sync_copy(zb, acc.at[pl.ds(s * zslice, zslice)])
    plsc.subcore_barrier()

    def body(g, carry):
        pltpu.sync_copy(ones_v, acc.at[dstv.at[g]], add=True)
        return carry

    lax.fori_loop(0, NECH, body, 0)
    plsc.subcore_barrier()
    pltpu.sync_copy(acc.at[pl.ds(s * zslice, zslice)],
                    out_hbm.at[c, pl.ds(s * zslice, zslice)])


SEG = 2000
NSEG = EPW // SEG
NCHS = SEG // ECH


@functools.partial(
    pl.kernel,
    mesh=_mesh,
    out_type=jax.ShapeDtypeStruct((NC, NPAD, DIN), jnp.float32),
    scratch_types=[
        pltpu.VMEM((SEG,), jnp.int32),
        pltpu.VMEM((NECH, ECH), jnp.int32),
        pltpu.VMEM((2, ECH, DIN), jnp.float32),
        pltpu.VMEM((16, DIN), jnp.float32),
        pltpu.VMEM_SHARED((NPAD, DIN), jnp.float32),
        pltpu.SemaphoreType.DMA,
    ],
)
def _gcn_msg(y_hbm, src_hbm, dst_hbm, out_hbm, srcv, dstv, gbuf, zbuf, acc,
             gsem):
    c = lax.axis_index("c")
    s = lax.axis_index("s")
    wid = c * NS + s
    zslice = NPAD // NS
    _zero_rows(zbuf, 16, DIN)
    for t in range(zslice // 16):
        pltpu.sync_copy(zbuf, acc.at[pl.ds(s * zslice + t * 16, 16)])
    pltpu.sync_copy(dst_hbm.at[wid], dstv)
    plsc.subcore_barrier()

    def seg_body(q, carry0):
        pltpu.sync_copy(src_hbm.at[pl.ds(wid * EPW + q * SEG, SEG)], srcv)
        pltpu.async_copy(y_hbm.at[srcv.at[pl.ds(0, ECH)]], gbuf.at[0], gsem)

        def body(gp, carry):
            for b in range(2):
                g = 2 * gp + b

                @pl.when(g + 1 < NCHS)
                def _():
                    pltpu.async_copy(
                        y_hbm.at[srcv.at[pl.ds((g + 1) * ECH, ECH)]],
                        gbuf.at[1 - b], gsem)

                pltpu.make_async_copy(
                    y_hbm.at[pl.ds(0, ECH)], gbuf.at[b], gsem).wait()
                pltpu.sync_copy(gbuf.at[b],
                                acc.at[dstv.at[q * NCHS + g]], add=True)
            return carry

        lax.fori_loop(0, NCHS // 2, body, 0)
        return carry0

    lax.fori_loop(0, NSEG, seg_body, 0)
    plsc.subcore_barrier()
    pltpu.sync_copy(acc.at[pl.ds(s * zslice, zslice)],
                    out_hbm.at[c, pl.ds(s * zslice, zslice)])


def _dense_hop(x, w):
    def body(x_ref, w_ref, o_ref):
        h = jnp.dot(x_ref[...] * (1.0 / K), w_ref[...],
                    preferred_element_type=jnp.float32)
        o_ref[...] = jnp.where(h > 0, h, 0.01 * h)

    return pl.pallas_call(
        body,
        grid=(NPAD // 256,),
        in_specs=[pl.BlockSpec((256, DIN), lambda i: (i, 0)),
                  pl.BlockSpec((DIN, D1), lambda i: (0, 0))],
        out_specs=pl.BlockSpec((256, D1), lambda i: (i, 0)),
        out_shape=jax.ShapeDtypeStruct((NPAD, D1), jnp.float32),
    )(x, w)


def _gcn_pre(deg_t, x):
    def body(d_ref, x_ref, y_ref, dv_ref):
        deg = jnp.maximum(d_ref[:, 0:1] + d_ref[:, 1:2], 1.0)
        dinv = 1.0 / jnp.sqrt(deg)
        dv_ref[...] = dinv
        y_ref[...] = x_ref[...] * dinv

    return pl.pallas_call(
        body,
        grid=(NPAD // 256,),
        in_specs=[pl.BlockSpec((256, NC), lambda i: (i, 0)),
                  pl.BlockSpec((256, DIN), lambda i: (i, 0))],
        out_specs=[pl.BlockSpec((256, DIN), lambda i: (i, 0)),
                   pl.BlockSpec((256, 1), lambda i: (i, 0))],
        out_shape=[jax.ShapeDtypeStruct((NPAD, DIN), jnp.float32),
                   jax.ShapeDtypeStruct((NPAD, 1), jnp.float32)],
    )(deg_t, x)


def _att_pair(h1, h2, a_ref):
    s1 = jnp.dot(h1, a_ref[...], preferred_element_type=jnp.float32)
    s2 = jnp.dot(h2, a_ref[...], preferred_element_type=jnp.float32)
    m = jnp.maximum(s1, s2)
    e1 = jnp.exp(s1 - m)
    e2 = jnp.exp(s2 - m)
    inv = 1.0 / (e1 + e2)
    return (e1 * inv) * h1 + (e2 * inv) * h2


def _post(parts, dinv, c1, c2, gcn_w, gcn_b, at2_w, at2_a, atb_w, atb_a):
    def body(gp_ref, dv_ref, c1_ref, c2_ref, gw_ref, gb_ref, aw_ref,
             aa_ref, bw_ref, ba_ref, o_ref):
        agg = (gp_ref[0] + gp_ref[1]) * dv_ref[...]
        g = jnp.maximum(
            jnp.dot(agg, gw_ref[...], preferred_element_type=jnp.float32)
            + gb_ref[...], 0.0)
        h1 = jnp.tanh(jnp.dot(c1_ref[...], aw_ref[...],
                              preferred_element_type=jnp.float32))
        h2 = jnp.tanh(jnp.dot(c2_ref[...], aw_ref[...],
                              preferred_element_type=jnp.float32))
        a1 = _att_pair(h1, h2, aa_ref)
        hb1 = jnp.tanh(jnp.dot(a1, bw_ref[...],
                               preferred_element_type=jnp.float32))
        hb2 = jnp.tanh(jnp.dot(g, bw_ref[...],
                               preferred_element_type=jnp.float32))
        o_ref[...] = _att_pair(hb1, hb2, ba_ref)

    return pl.pallas_call(
        body,
        grid=(NPAD // 256,),
        in_specs=[pl.BlockSpec((NC, 256, DIN), lambda i: (0, i, 0)),
                  pl.BlockSpec((256, 1), lambda i: (i, 0)),
                  pl.BlockSpec((256, D1), lambda i: (i, 0)),
                  pl.BlockSpec((256, D1), lambda i: (i, 0)),
                  pl.BlockSpec((DIN, D2), lambda i: (0, 0)),
                  pl.BlockSpec((1, D2), lambda i: (0, 0)),
                  pl.BlockSpec((D1, D2), lambda i: (0, 0)),
                  pl.BlockSpec((D2, 1), lambda i: (0, 0)),
                  pl.BlockSpec((D2, D2), lambda i: (0, 0)),
                  pl.BlockSpec((D2, 1), lambda i: (0, 0))],
        out_specs=pl.BlockSpec((256, D2), lambda i: (i, 0)),
        out_shape=jax.ShapeDtypeStruct((NPAD, D2), jnp.float32),
    )(parts, dinv, c1, c2, gcn_w, gcn_b, at2_w, at2_a, atb_w, atb_a)


def _score(mi, di):
    def body(a_ref, b_ref, o_ref):
        sc = lax.dot_general(a_ref[...], b_ref[...],
                             (((1,), (1,)), ((), ())),
                             preferred_element_type=jnp.float32)
        o_ref[...] = 1.0 / (1.0 + jnp.exp(-sc))

    return pl.pallas_call(
        body,
        grid=(5,),
        in_specs=[pl.BlockSpec((1000, D2), lambda i: (i, 0)),
                  pl.BlockSpec((M, D2), lambda i: (0, 0))],
        out_specs=pl.BlockSpec((1000, M), lambda i: (i, 0)),
        out_shape=jax.ShapeDtypeStruct((M, M), jnp.float32),
    )(mi, di)


def kernel(channel_1_k_neighbors, channel_2_k_neighbors,
           md_adj_withsl_edge_index, node_table, ch1_W, ch2_W, gcn_W, gcn_b,
           at2_W, at2_a, atb_W, atb_a):
    table = jnp.pad(node_table, ((0, NPAD - N), (0, 0)))
    idx1 = jnp.pad(channel_1_k_neighbors.astype(jnp.int32),
                   ((0, NPAD - N), (0, 0))).reshape(-1)
    idx2 = jnp.pad(channel_2_k_neighbors.astype(jnp.int32),
                   ((0, NPAD - N), (0, 0))).reshape(-1)
    src = md_adj_withsl_edge_index[0].astype(jnp.int32)
    dst3 = md_adj_withsl_edge_index[1].astype(jnp.int32).reshape(
        NW, NECH, ECH)

    h1 = table
    for i in range(3):
        h1 = _dense_hop(_hop_gather(h1, idx1), ch1_W[i])
    h2 = table
    for i in range(4):
        h2 = _dense_hop(_hop_gather(h2, idx2), ch2_W[i])

    degp = _deg_count(dst3)
    y, dinv = _gcn_pre(degp.T, table)
    parts = _gcn_msg(y, src, dst3)

    embed = _post(parts, dinv, h1, h2, gcn_W,
                  jnp.reshape(gcn_b, (1, D2)), at2_W,
                  jnp.reshape(at2_a, (D2, 1)), atb_W,
                  jnp.reshape(atb_a, (D2, 1)))
    return _score(embed[:M], embed[M:N])


# speedup vs baseline: 3.6462x; 3.5991x over previous
import functools

import jax
import jax.numpy as jnp
from jax import lax
from jax.experimental import pallas as pl
from jax.experimental.pallas import tpu as pltpu
from jax.experimental.pallas import tpu_sc as plsc

N = 10000
M = 5000
NPAD = 10240
K = 32
DIN = 128
D1 = 128
D2 = 64
E = 320000

NC = 2
NS = 16
NW = NC * NS
NPW = NPAD // NW
HALF = NPAD // NC
CHN = 2
CHK = CHN * K
NCHH = NPW // CHN
NBUF = 4
EPW = E // NW
ECH = 40
NECH = EPW // ECH

_mesh = plsc.VectorSubcoreMesh(core_axis_name="c", subcore_axis_name="s")


def _zero_rows(zbuf, rows, width):
    for r in range(rows):
        for v in range(width // 16):
            zbuf[r, pl.ds(v * 16, 16)] = jnp.zeros((16,), jnp.float32)


@functools.partial(
    pl.kernel,
    mesh=_mesh,
    out_type=jax.ShapeDtypeStruct((NPAD * DIN,), jnp.float32),
    scratch_types=[
        pltpu.VMEM_SHARED((NPAD, DIN), jnp.float32),
        pltpu.VMEM((NPW * K,), jnp.int32),
        pltpu.VMEM((2, K, DIN), jnp.float32),
        pltpu.VMEM((2, DIN), jnp.float32),
        pltpu.SemaphoreType.DMA,
        pltpu.SemaphoreType.DMA,
    ],
)
def _hop_gather(table_hbm, idx_hbm, out_hbm, tab_sh, idx_v, gbuf, wbuf,
                gsem, wsem):
    c = lax.axis_index("c")
    s = lax.axis_index("s")
    wid = c * NS + s
    t0 = s * (NPAD // NS)
    pltpu.sync_copy(table_hbm.at[pl.ds(t0, NPAD // NS)],
                    tab_sh.at[pl.ds(t0, NPAD // NS)])
    pltpu.sync_copy(idx_hbm.at[pl.ds(wid * NPW * K, NPW * K)], idx_v)
    plsc.subcore_barrier()
    for b in range(2):
        pltpu.async_copy(tab_sh.at[idx_v.at[pl.ds(b * K, K)]],
                         gbuf.at[b], gsem)

    def body(np_, carry):
        for b in range(2):
            n = 2 * np_ + b
            pltpu.make_async_copy(
                table_hbm.at[pl.ds(0, K)], gbuf.at[b], gsem).wait()

            @pl.when(n >= 2)
            def _():
                pltpu.make_async_copy(
                    wbuf.at[b], out_hbm.at[pl.ds(0, DIN)], wsem).wait()

            for v in range(DIN // 16):
                col = pl.ds(v * 16, 16)
                a0 = gbuf[b, 0, col]
                a1 = gbuf[b, 1, col]
                a2 = gbuf[b, 2, col]
                a3 = gbuf[b, 3, col]
                for k in range(4, K, 4):
                    a0 = a0 + gbuf[b, k + 0, col]
                    a1 = a1 + gbuf[b, k + 1, col]
                    a2 = a2 + gbuf[b, k + 2, col]
                    a3 = a3 + gbuf[b, k + 3, col]
                wbuf[b, col] = (a0 + a1) + (a2 + a3)
            pltpu.async_copy(
                wbuf.at[b],
                out_hbm.at[pl.ds((wid * NPW + n) * DIN, DIN)], wsem)

            @pl.when(n + 2 < NPW)
            def _():
                pltpu.async_copy(
                    tab_sh.at[idx_v.at[pl.ds((n + 2) * K, K)]],
                    gbuf.at[b], gsem)
        return carry

    lax.fori_loop(0, NPW // 2, body, 0)
    for _ in range(2):
        pltpu.make_async_copy(
            wbuf.at[0], out_hbm.at[pl.ds(0, DIN)], wsem).wait()


@functools.partial(
    pl.kernel,
    mesh=_mesh,
    out_type=jax.ShapeDtypeStruct((NC, NPAD), jnp.float32),
    scratch_types=[
        pltpu.VMEM((NECH, ECH), jnp.int32),
        pltpu.VMEM((ECH,), jnp.float32),
        pltpu.VMEM((NPAD // NS,), jnp.float32),
        pltpu.VMEM_SHARED((NPAD,), jnp.float32),
    ],
)
def _deg_count(dst_hbm, out_hbm, dstv, ones_v, zb, acc):
    c = lax.axis_index("c")
    s = lax.axis_index("s")
    wid = c * NS + s
    zslice = NPAD // NS
    pltpu.sync_copy(dst_hbm.at[wid], dstv)
    for v in range(ECH // 16):
        ones_v[pl.ds(v * 16, 16)] = jnp.ones((16,), jnp.float32)
    for v in range(zslice // 16):
        zb[pl.ds(v * 16, 16)] = jnp.zeros((16,), jnp.float32)
    pltpu.sync_copy(zb, acc.at[pl.ds(s * zslice, zslice)])
    plsc.subcore_barrier()

    def body(g, carry):
        pltpu.sync_copy(ones_v, acc.at[dstv.at[g]], add=True)
        return carry

    lax.fori_loop(0, NECH, body, 0)
    plsc.subcore_barrier()
    pltpu.sync_copy(acc.at[pl.ds(s * zslice, zslice)],
                    out_hbm.at[c, pl.ds(s * zslice, zslice)])


SEG = 2000
NSEG = EPW // SEG
NCHS = SEG // ECH


@functools.partial(
    pl.kernel,
    mesh=_mesh,
    out_type=jax.ShapeDtypeStruct((NC, NPAD, DIN), jnp.float32),
    scratch_types=[
        pltpu.VMEM((SEG,), jnp.int32),
        pltpu.VMEM((NECH, ECH), jnp.int32),
        pltpu.VMEM((2, ECH, DIN), jnp.float32),
        pltpu.VMEM((16, DIN), jnp.float32),
        pltpu.VMEM_SHARED((NPAD, DIN), jnp.float32),
        pltpu.SemaphoreType.DMA,
    ],
)
def _gcn_msg(y_hbm, src_hbm, dst_hbm, out_hbm, srcv, dstv, gbuf, zbuf, acc,
             gsem):
    c = lax.axis_index("c")
    s = lax.axis_index("s")
    wid = c * NS + s
    zslice = NPAD // NS
    _zero_rows(zbuf, 16, DIN)
    for t in range(zslice // 16):
        pltpu.sync_copy(zbuf, acc.at[pl.ds(s * zslice + t * 16, 16)])
    pltpu.sync_copy(dst_hbm.at[wid], dstv)
    plsc.subcore_barrier()

    def seg_body(q, carry0):
        pltpu.sync_copy(src_hbm.at[pl.ds(wid * EPW + q * SEG, SEG)], srcv)
        pltpu.async_copy(y_hbm.at[srcv.at[pl.ds(0, ECH)]], gbuf.at[0], gsem)

        def body(gp, carry):
            for b in range(2):
                g = 2 * gp + b

                @pl.when(g + 1 < NCHS)
                def _():
                    pltpu.async_copy(
                        y_hbm.at[srcv.at[pl.ds((g + 1) * ECH, ECH)]],
                        gbuf.at[1 - b], gsem)

                pltpu.make_async_copy(
                    y_hbm.at[pl.ds(0, ECH)], gbuf.at[b], gsem).wait()
                pltpu.sync_copy(gbuf.at[b],
                                acc.at[dstv.at[q * NCHS + g]], add=True)
            return carry

        lax.fori_loop(0, NCHS // 2, body, 0)
        return carry0

    lax.fori_loop(0, NSEG, seg_body, 0)
    plsc.subcore_barrier()
    pltpu.sync_copy(acc.at[pl.ds(s * zslice, zslice)],
                    out_hbm.at[c, pl.ds(s * zslice, zslice)])


def _dense_hop(x, w):
    def body(x_ref, w_ref, o_ref):
        h = jnp.dot(x_ref[...] * (1.0 / K), w_ref[...],
                    preferred_element_type=jnp.float32)
        o_ref[...] = jnp.where(h > 0, h, 0.01 * h)

    return pl.pallas_call(
        body,
        grid=(NPAD // 256,),
        in_specs=[pl.BlockSpec((256, DIN), lambda i: (i, 0)),
                  pl.BlockSpec((DIN, D1), lambda i: (0, 0))],
        out_specs=pl.BlockSpec((256, D1), lambda i: (i, 0)),
        out_shape=jax.ShapeDtypeStruct((NPAD, D1), jnp.float32),
    )(x, w)


def _gcn_pre(deg_t, x):
    def body(d_ref, x_ref, y_ref, dv_ref):
        deg = jnp.maximum(d_ref[:, 0:1] + d_ref[:, 1:2], 1.0)
        dinv = 1.0 / jnp.sqrt(deg)
        dv_ref[...] = dinv
        y_ref[...] = x_ref[...] * dinv

    return pl.pallas_call(
        body,
        grid=(NPAD // 256,),
        in_specs=[pl.BlockSpec((256, NC), lambda i: (i, 0)),
                  pl.BlockSpec((256, DIN), lambda i: (i, 0))],
        out_specs=[pl.BlockSpec((256, DIN), lambda i: (i, 0)),
                   pl.BlockSpec((256, 1), lambda i: (i, 0))],
        out_shape=[jax.ShapeDtypeStruct((NPAD, DIN), jnp.float32),
                   jax.ShapeDtypeStruct((NPAD, 1), jnp.float32)],
    )(deg_t, x)


def _att_pair(h1, h2, a_ref):
    s1 = jnp.dot(h1, a_ref[...], preferred_element_type=jnp.float32)
    s2 = jnp.dot(h2, a_ref[...], preferred_element_type=jnp.float32)
    m = jnp.maximum(s1, s2)
    e1 = jnp.exp(s1 - m)
    e2 = jnp.exp(s2 - m)
    inv = 1.0 / (e1 + e2)
    return (e1 * inv) * h1 + (e2 * inv) * h2


def _post(parts, dinv, c1, c2, gcn_w, gcn_b, at2_w, at2_a, atb_w, atb_a):
    def body(gp_ref, dv_ref, c1_ref, c2_ref, gw_ref, gb_ref, aw_ref,
             aa_ref, bw_ref, ba_ref, o_ref):
        agg = (gp_ref[0] + gp_ref[1]) * dv_ref[...]
        g = jnp.maximum(
            jnp.dot(agg, gw_ref[...], preferred_element_type=jnp.float32)
            + gb_ref[...], 0.0)
        h1 = jnp.tanh(jnp.dot(c1_ref[...], aw_ref[...],
                              preferred_element_type=jnp.float32))
        h2 = jnp.tanh(jnp.dot(c2_ref[...], aw_ref[...],
                              preferred_element_type=jnp.float32))
        a1 = _att_pair(h1, h2, aa_ref)
        hb1 = jnp.tanh(jnp.dot(a1, bw_ref[...],
                               preferred_element_type=jnp.float32))
        hb2 = jnp.tanh(jnp.dot(g, bw_ref[...],
                               preferred_element_type=jnp.float32))
        o_ref[...] = _att_pair(hb1, hb2, ba_ref)

    return pl.pallas_call(
        body,
        grid=(NPAD // 256,),
        in_specs=[pl.BlockSpec((NC, 256, DIN), lambda i: (0, i, 0)),
                  pl.BlockSpec((256, 1), lambda i: (i, 0)),
                  pl.BlockSpec((256, D1), lambda i: (i, 0)),
                  pl.BlockSpec((256, D1), lambda i: (i, 0)),
                  pl.BlockSpec((DIN, D2), lambda i: (0, 0)),
                  pl.BlockSpec((1, D2), lambda i: (0, 0)),
                  pl.BlockSpec((D1, D2), lambda i: (0, 0)),
                  pl.BlockSpec((D2, 1), lambda i: (0, 0)),
                  pl.BlockSpec((D2, D2), lambda i: (0, 0)),
                  pl.BlockSpec((D2, 1), lambda i: (0, 0))],
        out_specs=pl.BlockSpec((256, D2), lambda i: (i, 0)),
        out_shape=jax.ShapeDtypeStruct((NPAD, D2), jnp.float32),
    )(parts, dinv, c1, c2, gcn_w, gcn_b, at2_w, at2_a, atb_w, atb_a)


def _score(mi, di):
    def body(a_ref, b_ref, o_ref):
        sc = lax.dot_general(a_ref[...], b_ref[...],
                             (((1,), (1,)), ((), ())),
                             preferred_element_type=jnp.float32)
        o_ref[...] = 1.0 / (1.0 + jnp.exp(-sc))

    return pl.pallas_call(
        body,
        grid=(5,),
        in_specs=[pl.BlockSpec((1000, D2), lambda i: (i, 0)),
                  pl.BlockSpec((M, D2), lambda i: (0, 0))],
        out_specs=pl.BlockSpec((1000, M), lambda i: (i, 0)),
        out_shape=jax.ShapeDtypeStruct((M, M), jnp.float32),
    )(mi, di)


def kernel(channel_1_k_neighbors, channel_2_k_neighbors,
           md_adj_withsl_edge_index, node_table, ch1_W, ch2_W, gcn_W, gcn_b,
           at2_W, at2_a, atb_W, atb_a):
    table = jnp.pad(node_table, ((0, NPAD - N), (0, 0)))
    idx1 = jnp.pad(channel_1_k_neighbors.astype(jnp.int32),
                   ((0, NPAD - N), (0, 0))).reshape(-1)
    idx2 = jnp.pad(channel_2_k_neighbors.astype(jnp.int32),
                   ((0, NPAD - N), (0, 0))).reshape(-1)
    src = md_adj_withsl_edge_index[0].astype(jnp.int32)
    dst3 = md_adj_withsl_edge_index[1].astype(jnp.int32).reshape(
        NW, NECH, ECH)

    h1 = table
    for i in range(3):
        h1 = _dense_hop(_hop_gather(h1, idx1).reshape(NPAD, DIN), ch1_W[i])
    h2 = table
    for i in range(4):
        h2 = _dense_hop(_hop_gather(h2, idx2).reshape(NPAD, DIN), ch2_W[i])

    degp = _deg_count(dst3)
    y, dinv = _gcn_pre(degp.T, table)
    parts = _gcn_msg(y, src, dst3)

    embed = _post(parts, dinv, h1, h2, gcn_W,
                  jnp.reshape(gcn_b, (1, D2)), at2_W,
                  jnp.reshape(at2_a, (D2, 1)), atb_W,
                  jnp.reshape(atb_a, (D2, 1)))
    return _score(embed[:M], embed[M:N])
